# baseline (speedup 1.0000x reference)
import functools

import jax
import jax.numpy as jnp
from jax import lax
from jax.experimental import pallas as pl
from jax.experimental.pallas import tpu as pltpu
from jax.experimental.pallas import tpu_sc as plsc

BATCH = 16384
EMB = 64
NUM_WORKERS = 32
NB = 7813
RPT = 245
NTRIP = 246
KMAX = 640
NSLOT = 3
OUTROWS = BATCH + 128

_sc_mesh = plsc.VectorSubcoreMesh(core_axis_name="c", subcore_axis_name="s")


@functools.partial(
    pl.kernel,
    mesh=_sc_mesh,
    compiler_params=pltpu.CompilerParams(needs_layout_passes=False),
    out_type=jax.ShapeDtypeStruct((OUTROWS, 2 * EMB), jnp.float32),
    scratch_types=[
        pltpu.VMEM((32, 128), jnp.int32),
        pltpu.VMEM((KMAX,), jnp.int32),
        pltpu.VMEM((KMAX,), jnp.int32),
        pltpu.VMEM((KMAX // 128, 128), jnp.int32),
        pltpu.VMEM((NSLOT, 8, 8, 128), jnp.float32),
        pltpu.VMEM((KMAX, 2 * EMB), jnp.float32),
        pltpu.SemaphoreType.DMA,
    ],
)
def _sc_block_gather(ids_hbm, tab3_hbm, out_hbm, chunk_v, kid_v, kpos1_v,
                     kpos_v, slab_v, rows_v, sem):
    wid = lax.axis_index("s") * 2 + lax.axis_index("c")
    lo = wid * RPT
    nblk = jnp.minimum(lo + RPT, NB) - lo
    iota = lax.iota(jnp.int32, 16)

    def scan_chunk(c, cnt0):
        pltpu.sync_copy(ids_hbm.at[pl.ds(c * 32, 32)], chunk_v)

        def row_body(r, cnt1):
            cnt2 = cnt1
            for s in range(8):
                ids16 = chunk_v.at[r][pl.ds(16 * s, 16)]
                rb = ids16 >> 7
                m = (rb >= lo) & (rb < lo + nblk)
                pos = c * 4096 + r * 128 + 16 * s + iota
                prefix = plsc.cumsum(m.astype(jnp.int32))
                dest = jnp.broadcast_to(cnt2, (16,)) + prefix - 1
                plsc.store_scatter(kid_v, [dest], ids16, mask=m)
                plsc.store_scatter(kpos1_v, [dest], pos, mask=m)
                cnt2 = cnt2 + jnp.sum(m.astype(jnp.int32), axis=0)
            return cnt2

        return lax.fori_loop(0, 32, row_body, cnt0)

    cnt = lax.fori_loop(0, 4, scan_chunk, jnp.int32(0))

    for j in range(KMAX // 16):
        lane = j * 16 + iota
        p16 = kpos1_v.at[pl.ds(j * 16, 16)][...]
        pos16 = jnp.where(lane < cnt, p16, BATCH + (lane & 127))
        kpos_v.at[j // 8].at[pl.ds((j % 8) * 16, 16)][...] = pos16

    def issue(g, b):
        ge = jnp.minimum(g, nblk - 1)
        off = pl.multiple_of((lo + ge) * 128, 128)
        pltpu.async_copy(
            tab3_hbm.at[:, :, pl.ds(off, 128)], slab_v.at[b], sem)

    def drain(b):
        pltpu.make_async_copy(
            tab3_hbm.at[:, :, pl.ds(0, 128)], slab_v.at[b], sem).wait()

    def extract(g, b):
        ge = jnp.minimum(g, nblk - 1)
        rbg = lo + ge
        slab = slab_v.at[b]

        @pl.loop(0, KMAX // 16)
        def _(j):
            lane = j * 16 + iota
            ids16 = kid_v.at[pl.ds(j * 16, 16)][...]
            m = ((ids16 >> 7) == rbg) & (lane < cnt)

            @pl.when(jnp.any(m))
            def _():
                rl16 = ids16 & 127
                for c in range(EMB):
                    val = plsc.load_gather(
                        slab,
                        [jnp.broadcast_to(c >> 3, (16,)),
                         jnp.broadcast_to(c & 7, (16,)), rl16],
                        mask=m)
                    plsc.store_scatter(
                        rows_v, [lane, jnp.broadcast_to(c, (16,))], val,
                        mask=m)

    for b in range(NSLOT):
        issue(b, b)

    @pl.loop(0, NTRIP // NSLOT - 1)
    def _(go):
        for b in range(NSLOT):
            g = go * NSLOT + b
            drain(b)
            extract(g, b)
            issue(g + NSLOT, b)

    for b in range(NSLOT):
        g = NTRIP - NSLOT + b
        drain(b)
        extract(g, b)

    for q in range(KMAX // 128):
        pltpu.sync_copy(rows_v.at[pl.ds(q * 128, 128)],
                        out_hbm.at[kpos_v.at[q]])


def _mlp_body(u_ref, i_ref, w1_ref, b1_ref, w2_ref, b2_ref, o_ref):
    prod = u_ref[:, 0:EMB] * i_ref[:, 0:EMB]
    h = jnp.dot(prod, w1_ref[...], preferred_element_type=jnp.float32)
    h = jnp.maximum(h + b1_ref[...], 0.0)
    o = jnp.sum(h * w2_ref[...], axis=1) + b2_ref[0, 0]
    o_ref[...] = jax.nn.sigmoid(o)


def kernel(user_ids, item_ids, user_table, item_table, W1, b1, W2, b2):
    uid = user_ids.astype(jnp.int32).reshape(128, 128)
    iid = item_ids.astype(jnp.int32).reshape(128, 128)
    ut3 = user_table.T.reshape(8, 8, user_table.shape[0])
    it3 = item_table.T.reshape(8, 8, item_table.shape[0])

    u_emb = _sc_block_gather(uid, ut3)
    i_emb = _sc_block_gather(iid, it3)

    blk = 2048
    out = pl.pallas_call(
        _mlp_body,
        grid=(BATCH // blk,),
        in_specs=[
            pl.BlockSpec((blk, 2 * EMB), lambda b: (b, 0)),
            pl.BlockSpec((blk, 2 * EMB), lambda b: (b, 0)),
            pl.BlockSpec((EMB, 32), lambda b: (0, 0)),
            pl.BlockSpec((1, 32), lambda b: (0, 0)),
            pl.BlockSpec((1, 32), lambda b: (0, 0)),
            pl.BlockSpec((1, 1), lambda b: (0, 0)),
        ],
        out_specs=pl.BlockSpec((blk,), lambda b: (b,)),
        out_shape=jax.ShapeDtypeStruct((BATCH,), jnp.float32),
    )(u_emb, i_emb, W1, b1.reshape(1, 32), W2.reshape(1, 32),
      b2.reshape(1, 1))
    return out


# speedup vs baseline: 2.0445x; 2.0445x over previous
import functools

import jax
import jax.numpy as jnp
from jax import lax
from jax.experimental import pallas as pl
from jax.experimental.pallas import tpu as pltpu
from jax.experimental.pallas import tpu_sc as plsc

BATCH = 16384
EMB = 64
NUM_WORKERS = 32
NB = 7813
RPT = 245
NTRIP = 246
KMAX = 640
BCAP = 112
NSLOT = 3
OUTROWS = BATCH + 128

_sc_mesh = plsc.VectorSubcoreMesh(core_axis_name="c", subcore_axis_name="s")


@functools.partial(
    pl.kernel,
    mesh=_sc_mesh,
    compiler_params=pltpu.CompilerParams(needs_layout_passes=False),
    out_type=jax.ShapeDtypeStruct((OUTROWS, 2 * EMB), jnp.float32),
    scratch_types=[
        pltpu.VMEM((32, 128), jnp.int32),
        pltpu.VMEM((KMAX,), jnp.int32),
        pltpu.VMEM((KMAX,), jnp.int32),
        pltpu.VMEM((KMAX // 128, 128), jnp.int32),
        pltpu.VMEM((8, BCAP), jnp.int32),
        pltpu.VMEM((8, BCAP), jnp.int32),
        pltpu.VMEM((16,), jnp.int32),
        pltpu.VMEM((NSLOT, 8, 8, 128), jnp.float32),
        pltpu.VMEM((KMAX, 2 * EMB), jnp.float32),
        pltpu.SemaphoreType.DMA,
    ],
)
def _sc_block_gather(ids_hbm, tab3_hbm, out_hbm, chunk_v, kid_v, kpos1_v,
                     kpos_v, bid_v, bk_v, bcnt_v, slab_v, rows_v, sem):
    wid = lax.axis_index("s") * 2 + lax.axis_index("c")
    lo = wid * RPT
    nblk = jnp.minimum(lo + RPT, NB) - lo
    iota = lax.iota(jnp.int32, 16)

    def scan_chunk(c, cnt0):
        pltpu.sync_copy(ids_hbm.at[pl.ds(c * 32, 32)], chunk_v)

        def row_body(r, cnt1):
            cnt2 = cnt1
            for s in range(8):
                ids16 = chunk_v.at[r][pl.ds(16 * s, 16)]
                rb = ids16 >> 7
                m = (rb >= lo) & (rb < lo + nblk)
                pos = c * 4096 + r * 128 + 16 * s + iota
                prefix = plsc.cumsum(m.astype(jnp.int32))
                dest = jnp.broadcast_to(cnt2, (16,)) + prefix - 1
                plsc.store_scatter(kid_v, [dest], ids16, mask=m)
                plsc.store_scatter(kpos1_v, [dest], pos, mask=m)
                cnt2 = cnt2 + jnp.sum(m.astype(jnp.int32), axis=0)
            return cnt2

        return lax.fori_loop(0, 32, row_body, cnt0)

    cnt = lax.fori_loop(0, 4, scan_chunk, jnp.int32(0))

    for j in range(KMAX // 16):
        lane = j * 16 + iota
        p16 = kpos1_v.at[pl.ds(j * 16, 16)][...]
        pos16 = jnp.where(lane < cnt, p16, BATCH + (lane & 127))
        kpos_v.at[j // 8].at[pl.ds((j % 8) * 16, 16)][...] = pos16

    for sub in range(8):
        def bucket_body(j, bc, _sub=sub):
            lane = j * 16 + iota
            ids16 = kid_v.at[pl.ds(j * 16, 16)][...]
            sub16 = (ids16 >> 7) - lo
            m = ((sub16 >> 5) == _sub) & (lane < cnt)
            prefix = plsc.cumsum(m.astype(jnp.int32))
            dest = jnp.broadcast_to(bc, (16,)) + prefix - 1
            plsc.store_scatter(bid_v.at[_sub], [dest], ids16, mask=m)
            plsc.store_scatter(bk_v.at[_sub], [dest], lane, mask=m)
            return bc + jnp.sum(m.astype(jnp.int32), axis=0)

        bc = lax.fori_loop(0, KMAX // 16, bucket_body, jnp.int32(0))
        plsc.store_scatter(bcnt_v, [jnp.broadcast_to(sub, (16,))],
                           jnp.broadcast_to(bc, (16,)), mask=(iota == 0))

    def issue(g, b):
        ge = jnp.minimum(g, nblk - 1)
        off = pl.multiple_of((lo + ge) * 128, 128)
        pltpu.async_copy(
            tab3_hbm.at[:, :, pl.ds(off, 128)], slab_v.at[b], sem)

    def drain(b):
        pltpu.make_async_copy(
            tab3_hbm.at[:, :, pl.ds(0, 128)], slab_v.at[b], sem).wait()

    def extract(g, b):
        ge = jnp.minimum(g, nblk - 1)
        rbg = lo + ge
        sub = (ge >> 5).astype(jnp.int32)
        bcnt = plsc.load_gather(bcnt_v, [jnp.broadcast_to(sub, (16,))])
        slab = slab_v.at[b]

        @pl.loop(0, BCAP // 16)
        def _(t):
            lane = t * 16 + iota
            ids16 = bid_v.at[sub].at[pl.ds(t * 16, 16)][...]
            m = ((ids16 >> 7) == rbg) & (lane < bcnt)

            @pl.when(jnp.any(m))
            def _(_m=m, _ids=ids16, _sub=sub, _t=t):
                k16 = bk_v.at[_sub].at[pl.ds(_t * 16, 16)][...]
                rl16 = _ids & 127
                for c in range(EMB):
                    val = plsc.load_gather(
                        slab,
                        [jnp.broadcast_to(c >> 3, (16,)),
                         jnp.broadcast_to(c & 7, (16,)), rl16],
                        mask=_m)
                    plsc.store_scatter(
                        rows_v, [k16, jnp.broadcast_to(c, (16,))], val,
                        mask=_m)

    for b in range(NSLOT):
        issue(b, b)

    @pl.loop(0, NTRIP // NSLOT - 1)
    def _(go):
        for b in range(NSLOT):
            g = go * NSLOT + b
            drain(b)
            extract(g, b)
            issue(g + NSLOT, b)

    for b in range(NSLOT):
        g = NTRIP - NSLOT + b
        drain(b)
        extract(g, b)

    for q in range(KMAX // 128):
        pltpu.sync_copy(rows_v.at[pl.ds(q * 128, 128)],
                        out_hbm.at[kpos_v.at[q]])


def _mlp_body(u_ref, i_ref, w1_ref, b1_ref, w2_ref, b2_ref, o_ref):
    prod = u_ref[:, 0:EMB] * i_ref[:, 0:EMB]
    h = jnp.dot(prod, w1_ref[...], preferred_element_type=jnp.float32)
    h = jnp.maximum(h + b1_ref[...], 0.0)
    o = jnp.sum(h * w2_ref[...], axis=1) + b2_ref[0, 0]
    o_ref[...] = jax.nn.sigmoid(o)


def kernel(user_ids, item_ids, user_table, item_table, W1, b1, W2, b2):
    uid = user_ids.astype(jnp.int32).reshape(128, 128)
    iid = item_ids.astype(jnp.int32).reshape(128, 128)
    ut3 = user_table.T.reshape(8, 8, user_table.shape[0])
    it3 = item_table.T.reshape(8, 8, item_table.shape[0])

    u_emb = _sc_block_gather(uid, ut3)
    i_emb = _sc_block_gather(iid, it3)

    blk = 2048
    out = pl.pallas_call(
        _mlp_body,
        grid=(BATCH // blk,),
        in_specs=[
            pl.BlockSpec((blk, 2 * EMB), lambda b: (b, 0)),
            pl.BlockSpec((blk, 2 * EMB), lambda b: (b, 0)),
            pl.BlockSpec((EMB, 32), lambda b: (0, 0)),
            pl.BlockSpec((1, 32), lambda b: (0, 0)),
            pl.BlockSpec((1, 32), lambda b: (0, 0)),
            pl.BlockSpec((1, 1), lambda b: (0, 0)),
        ],
        out_specs=pl.BlockSpec((blk,), lambda b: (b,)),
        out_shape=jax.ShapeDtypeStruct((BATCH,), jnp.float32),
    )(u_emb, i_emb, W1, b1.reshape(1, 32), W2.reshape(1, 32),
      b2.reshape(1, 1))
    return out


# speedup vs baseline: 2.1367x; 1.0451x over previous
import functools

import jax
import jax.numpy as jnp
from jax import lax
from jax.experimental import pallas as pl
from jax.experimental.pallas import tpu as pltpu
from jax.experimental.pallas import tpu_sc as plsc

BATCH = 16384
EMB = 64
NUM_WORKERS = 32
NB = 7813
RPT = 245
NTRIP = 248
KMAX = 640
BCAP = 112
NSLOT = 4
OUTROWS = BATCH + 128

_sc_mesh = plsc.VectorSubcoreMesh(core_axis_name="c", subcore_axis_name="s")


@functools.partial(
    pl.kernel,
    mesh=_sc_mesh,
    compiler_params=pltpu.CompilerParams(needs_layout_passes=False),
    out_type=jax.ShapeDtypeStruct((OUTROWS, 2 * EMB), jnp.float32),
    scratch_types=[
        pltpu.VMEM((32, 128), jnp.int32),
        pltpu.VMEM((KMAX,), jnp.int32),
        pltpu.VMEM((KMAX,), jnp.int32),
        pltpu.VMEM((KMAX // 128, 128), jnp.int32),
        pltpu.VMEM((8, BCAP), jnp.int32),
        pltpu.VMEM((8, BCAP), jnp.int32),
        pltpu.VMEM((16,), jnp.int32),
        pltpu.VMEM((NSLOT, 8, 8, 128), jnp.float32),
        pltpu.VMEM((KMAX, 2 * EMB), jnp.float32),
        pltpu.SemaphoreType.DMA,
    ],
)
def _sc_block_gather(ids_hbm, tab3_hbm, out_hbm, chunk_v, kid_v, kpos1_v,
                     kpos_v, bid_v, bk_v, bcnt_v, slab_v, rows_v, sem):
    wid = lax.axis_index("s") * 2 + lax.axis_index("c")
    lo = wid * RPT
    nblk = jnp.minimum(lo + RPT, NB) - lo
    iota = lax.iota(jnp.int32, 16)

    def scan_chunk(c, cnt0):
        pltpu.sync_copy(ids_hbm.at[pl.ds(c * 32, 32)], chunk_v)

        def row_body(r, cnt1):
            cnt2 = cnt1
            for s in range(8):
                ids16 = chunk_v.at[r][pl.ds(16 * s, 16)]
                rb = ids16 >> 7
                m = (rb >= lo) & (rb < lo + nblk)
                pos = c * 4096 + r * 128 + 16 * s + iota
                prefix = plsc.cumsum(m.astype(jnp.int32))
                dest = jnp.broadcast_to(cnt2, (16,)) + prefix - 1
                plsc.store_scatter(kid_v, [dest], ids16, mask=m)
                plsc.store_scatter(kpos1_v, [dest], pos, mask=m)
                cnt2 = cnt2 + jnp.sum(m.astype(jnp.int32), axis=0)
            return cnt2

        return lax.fori_loop(0, 32, row_body, cnt0)

    cnt = lax.fori_loop(0, 4, scan_chunk, jnp.int32(0))

    for j in range(KMAX // 16):
        lane = j * 16 + iota
        p16 = kpos1_v.at[pl.ds(j * 16, 16)][...]
        pos16 = jnp.where(lane < cnt, p16, BATCH + (lane & 127))
        kpos_v.at[j // 8].at[pl.ds((j % 8) * 16, 16)][...] = pos16

    for sub in range(8):
        def bucket_body(j, bc, _sub=sub):
            lane = j * 16 + iota
            ids16 = kid_v.at[pl.ds(j * 16, 16)][...]
            sub16 = (ids16 >> 7) - lo
            m = ((sub16 >> 5) == _sub) & (lane < cnt)
            prefix = plsc.cumsum(m.astype(jnp.int32))
            dest = jnp.broadcast_to(bc, (16,)) + prefix - 1
            plsc.store_scatter(bid_v.at[_sub], [dest], ids16, mask=m)
            plsc.store_scatter(bk_v.at[_sub], [dest], lane, mask=m)
            return bc + jnp.sum(m.astype(jnp.int32), axis=0)

        bc = lax.fori_loop(0, KMAX // 16, bucket_body, jnp.int32(0))
        plsc.store_scatter(bcnt_v, [jnp.broadcast_to(sub, (16,))],
                           jnp.broadcast_to(bc, (16,)), mask=(iota == 0))

    def issue(g, b):
        ge = jnp.minimum(g, nblk - 1)
        off = pl.multiple_of((lo + ge) * 128, 128)
        pltpu.async_copy(
            tab3_hbm.at[:, :, pl.ds(off, 128)], slab_v.at[b], sem)

    def drain(b):
        pltpu.make_async_copy(
            tab3_hbm.at[:, :, pl.ds(0, 128)], slab_v.at[b], sem).wait()

    def extract(g, b):
        ge = jnp.minimum(g, nblk - 1)
        rbg = lo + ge
        sub = (ge >> 5).astype(jnp.int32)
        bcnt = plsc.load_gather(bcnt_v, [jnp.broadcast_to(sub, (16,))])
        slab = slab_v.at[b]

        @pl.loop(0, BCAP // 16)
        def _(t):
            lane = t * 16 + iota
            ids16 = bid_v.at[sub].at[pl.ds(t * 16, 16)][...]
            m = ((ids16 >> 7) == rbg) & (lane < bcnt)

            @pl.when(jnp.any(m))
            def _(_m=m, _ids=ids16, _sub=sub, _t=t):
                k16 = bk_v.at[_sub].at[pl.ds(_t * 16, 16)][...]
                rl16 = _ids & 127
                for c in range(EMB):
                    val = plsc.load_gather(
                        slab,
                        [jnp.broadcast_to(c >> 3, (16,)),
                         jnp.broadcast_to(c & 7, (16,)), rl16],
                        mask=_m)
                    plsc.store_scatter(
                        rows_v, [k16, jnp.broadcast_to(c, (16,))], val,
                        mask=_m)

    for b in range(NSLOT):
        issue(b, b)

    @pl.loop(0, NTRIP // NSLOT - 1)
    def _(go):
        for b in range(NSLOT):
            g = go * NSLOT + b
            drain(b)
            extract(g, b)
            issue(g + NSLOT, b)

    for b in range(NSLOT):
        g = NTRIP - NSLOT + b
        drain(b)
        extract(g, b)

    for q in range(KMAX // 128):
        pltpu.sync_copy(rows_v.at[pl.ds(q * 128, 128)],
                        out_hbm.at[kpos_v.at[q]])


def _mlp_body(u_ref, i_ref, w1_ref, b1_ref, w2_ref, b2_ref, o_ref):
    prod = u_ref[:, 0:EMB] * i_ref[:, 0:EMB]
    h = jnp.dot(prod, w1_ref[...], preferred_element_type=jnp.float32)
    h = jnp.maximum(h + b1_ref[...], 0.0)
    o = jnp.sum(h * w2_ref[...], axis=1) + b2_ref[0, 0]
    o_ref[...] = jax.nn.sigmoid(o)


def kernel(user_ids, item_ids, user_table, item_table, W1, b1, W2, b2):
    uid = user_ids.astype(jnp.int32).reshape(128, 128)
    iid = item_ids.astype(jnp.int32).reshape(128, 128)
    ut3 = user_table.T.reshape(8, 8, user_table.shape[0])
    it3 = item_table.T.reshape(8, 8, item_table.shape[0])

    u_emb = _sc_block_gather(uid, ut3)
    i_emb = _sc_block_gather(iid, it3)

    blk = 2048
    out = pl.pallas_call(
        _mlp_body,
        grid=(BATCH // blk,),
        in_specs=[
            pl.BlockSpec((blk, 2 * EMB), lambda b: (b, 0)),
            pl.BlockSpec((blk, 2 * EMB), lambda b: (b, 0)),
            pl.BlockSpec((EMB, 32), lambda b: (0, 0)),
            pl.BlockSpec((1, 32), lambda b: (0, 0)),
            pl.BlockSpec((1, 32), lambda b: (0, 0)),
            pl.BlockSpec((1, 1), lambda b: (0, 0)),
        ],
        out_specs=pl.BlockSpec((blk,), lambda b: (b,)),
        out_shape=jax.ShapeDtypeStruct((BATCH,), jnp.float32),
    )(u_emb, i_emb, W1, b1.reshape(1, 32), W2.reshape(1, 32),
      b2.reshape(1, 1))
    return out
